# Initial kernel scaffold; baseline (speedup 1.0000x reference)
import jax
import jax.numpy as jnp
from jax.experimental import pallas as pl


def kernel(feat_index, feat_value, emb_table, W1, b1, W2, b2, Wo, bo):
    raise NotImplementedError("write your pallas kernel here")


# speedup vs baseline: 1.1806x; 1.1806x over previous
import functools

import jax
import jax.numpy as jnp
from jax import lax
from jax.experimental import pallas as pl
from jax.experimental.pallas import tpu as pltpu
from jax.experimental.pallas import tpu_sc as plsc

B = 16384
F = 26
E = 16
NC = 2
NS = 16
NW = NC * NS
BPW = B // NW
CHUNK = 64
NCHUNK = BPW // CHUNK
RPC = CHUNK * F
GW = 128
NG = RPC // GW
NIDX = BPW * F // GW


def _sc_body(table, idx_hbm, val_hbm, out_hbm,
             idx_v, val_v, rows_a, rows_b, out_v, sem_a, sem_b):
    c = lax.axis_index("c")
    s = lax.axis_index("s")
    wid = s * NC + c

    pltpu.sync_copy(idx_hbm.at[wid], idx_v)
    pltpu.sync_copy(val_hbm.at[wid], val_v)

    bufs = (rows_a, rows_b)
    sems = (sem_a, sem_b)

    def fire(chunk, slot):
        handles = []
        for j in range(NG):
            h = pltpu.make_async_copy(
                table.at[idx_v.at[chunk * NG + j]],
                bufs[slot].at[pl.ds(j * GW, GW), :],
                sems[slot])
            h.start()
            handles.append(h)
        return handles

    def compute(chunk, slot):
        rows = bufs[slot]

        def body(i, _):
            b = chunk * CHUNK + i
            r0 = i * F
            v_lo = val_v[b, pl.ds(0, E)]
            v_hi = val_v[b, pl.ds(E, E)]
            acc0 = jnp.zeros((E,), jnp.float32)
            acc1 = jnp.zeros((E,), jnp.float32)
            sq0 = jnp.zeros((E,), jnp.float32)
            sq1 = jnp.zeros((E,), jnp.float32)
            for f in range(F):
                row = rows[r0 + f, :]
                scalar = v_lo[f] if f < E else v_hi[f - E]
                wv = row * jnp.broadcast_to(scalar, (E,))
                if f % 2 == 0:
                    acc0 = acc0 + wv
                    sq0 = sq0 + wv * wv
                else:
                    acc1 = acc1 + wv
                    sq1 = sq1 + wv * wv
            acc = acc0 + acc1
            sq = sq0 + sq1
            out_v[b, :] = (acc * acc - sq) * 0.5
            return ()

        lax.fori_loop(0, CHUNK, body, (), unroll=False)

    pending = fire(0, 0)
    for chunk in range(NCHUNK):
        slot = chunk % 2
        for h in pending:
            h.wait()
        if chunk + 1 < NCHUNK:
            pending = fire(chunk + 1, 1 - slot)
        compute(chunk, slot)

    pltpu.sync_copy(out_v, out_hbm.at[pl.ds(wid * BPW, BPW), :])


@jax.jit
def _bi_interaction_sc(feat_index, feat_value, emb_table):
    idx = feat_index.reshape(NW, NIDX, GW).astype(jnp.int32)
    val = jnp.pad(feat_value, ((0, 0), (0, 2 * E - F))).reshape(NW, BPW, 2 * E)
    mesh = plsc.VectorSubcoreMesh(core_axis_name="c", subcore_axis_name="s")
    fn = pl.kernel(
        _sc_body,
        out_type=jax.ShapeDtypeStruct((B, E), jnp.float32),
        mesh=mesh,
        compiler_params=pltpu.CompilerParams(use_tc_tiling_on_sc=False),
        scratch_types=[
            pltpu.VMEM((NIDX, GW), jnp.int32),
            pltpu.VMEM((BPW, 2 * E), jnp.float32),
            pltpu.VMEM((RPC, E), jnp.float32),
            pltpu.VMEM((RPC, E), jnp.float32),
            pltpu.VMEM((BPW, E), jnp.float32),
            pltpu.SemaphoreType.DMA,
            pltpu.SemaphoreType.DMA,
        ],
    )
    return fn(emb_table, idx, val)


def _mlp_body(bi_ref, w1_ref, b1_ref, w2_ref, b2_ref, wo_ref, bo_ref, out_ref):
    x = bi_ref[...]
    h = jnp.dot(x, w1_ref[...], preferred_element_type=jnp.float32)
    h = jnp.maximum(h + b1_ref[...], 0.0)
    h = jnp.dot(h, w2_ref[...], preferred_element_type=jnp.float32)
    h = jnp.maximum(h + b2_ref[...], 0.0)
    o = jnp.sum(h * wo_ref[...], axis=1, keepdims=True) + bo_ref[...]
    out_ref[...] = 1.0 / (1.0 + jnp.exp(-o))


@jax.jit
def _mlp_tc(bi, W1, b1, W2, b2, Wo, bo):
    nblk = 8
    blk = B // nblk
    return pl.pallas_call(
        _mlp_body,
        grid=(nblk,),
        in_specs=[
            pl.BlockSpec((blk, E), lambda i: (i, 0)),
            pl.BlockSpec((E, 32), lambda i: (0, 0)),
            pl.BlockSpec((1, 32), lambda i: (0, 0)),
            pl.BlockSpec((32, 32), lambda i: (0, 0)),
            pl.BlockSpec((1, 32), lambda i: (0, 0)),
            pl.BlockSpec((1, 32), lambda i: (0, 0)),
            pl.BlockSpec((1, 1), lambda i: (0, 0)),
        ],
        out_specs=pl.BlockSpec((blk, 1), lambda i: (i, 0)),
        out_shape=jax.ShapeDtypeStruct((B, 1), jnp.float32),
    )(bi, W1, b1.reshape(1, 32), W2, b2.reshape(1, 32),
      Wo.reshape(1, 32), bo.reshape(1, 1))


def kernel(feat_index, feat_value, emb_table, W1, b1, W2, b2, Wo, bo):
    bi = _bi_interaction_sc(feat_index, feat_value, emb_table)
    return _mlp_tc(bi, W1, b1, W2, b2, Wo, bo)
